# Initial kernel scaffold; baseline (speedup 1.0000x reference)
import functools

import jax
import jax.numpy as jnp
from jax import lax
from jax.experimental import pallas as pl
from jax.experimental.pallas import tpu as pltpu
from jax.experimental.pallas import tpu_sc as plsc

N = 10000
E = 160000
D = 128
N_REL = 13
NTAB = N_REL + 1

NC = 2
NS = 16
NW = NC * NS
K = 128
EPT = 5120
EP = NW * EPT
C = EPT // K
NPAD = 10240
ROWS_PER_TILE_ACC = NPAD // NS
ROWS_PER_TILE_OUT = N // NS

BN = 500
GRID_N = N // BN


def _table_body(x_ref, w_ref, b_ref, bases_ref, coef_ref, out_ref):
    x = x_ref[...]
    out_ref[:, 0, :] = (
        jnp.dot(x, w_ref[...], preferred_element_type=jnp.float32) + b_ref[...]
    )
    h0 = jnp.dot(x, bases_ref[0], preferred_element_type=jnp.float32)
    h1 = jnp.dot(x, bases_ref[1], preferred_element_type=jnp.float32)
    for r in range(N_REL):
        out_ref[:, r + 1, :] = coef_ref[r, 0] * h0 + coef_ref[r, 1] * h1


def _build_table(x, W_self, b_self, bases, coefficients):
    return pl.pallas_call(
        _table_body,
        grid=(GRID_N,),
        in_specs=[
            pl.BlockSpec((BN, D), lambda i: (i, 0)),
            pl.BlockSpec((D, D), lambda i: (0, 0)),
            pl.BlockSpec((1, D), lambda i: (0, 0)),
            pl.BlockSpec((2, D, D), lambda i: (0, 0, 0)),
            pl.BlockSpec(memory_space=pltpu.SMEM),
        ],
        out_specs=pl.BlockSpec((BN, NTAB, D), lambda i: (i, 0, 0)),
        out_shape=jax.ShapeDtypeStruct((N, NTAB, D), jnp.float32),
    )(x, W_self, b_self.reshape(1, D), bases, coefficients)


def _sc_body(htab_hbm, src_hbm, et_hbm, tgt_hbm, out_hbm,
             idx_v, tmp_v, tgt_v, rows_v, acc, sem_g, sem_s):
    core = lax.axis_index("c")
    sub = lax.axis_index("s")
    wid = core * NS + sub

    @pl.loop(0, K)
    def _(i):
        for j in range(D // 16):
            rows_v[0, i, pl.ds(j * 16, 16)] = jnp.zeros((16,), jnp.float32)

    for k in range(ROWS_PER_TILE_ACC // K):
        pltpu.sync_copy(rows_v.at[0],
                        acc.at[pl.ds(sub * ROWS_PER_TILE_ACC + k * K, K)])

    pltpu.sync_copy(src_hbm.at[pl.ds(wid * C, C)], idx_v)
    pltpu.sync_copy(et_hbm.at[pl.ds(wid * C, C)], tmp_v)
    pltpu.sync_copy(tgt_hbm.at[pl.ds(wid * C, C)], tgt_v)

    @pl.loop(0, C)
    def _(c):
        for j in range(K // 16):
            s = idx_v[c, pl.ds(j * 16, 16)]
            t = tmp_v[c, pl.ds(j * 16, 16)]
            idx_v[c, pl.ds(j * 16, 16)] = s * NTAB + t + 1

    plsc.subcore_barrier()

    @pl.loop(0, C)
    def _(c):
        pltpu.async_copy(htab_hbm.at[idx_v.at[c]], rows_v.at[0],
                         sem_g.at[0]).wait()
        pltpu.async_copy(rows_v.at[0], acc.at[tgt_v.at[c]],
                         sem_s.at[0], add=True).wait()

    plsc.subcore_barrier()

    pltpu.sync_copy(acc.at[pl.ds(sub * ROWS_PER_TILE_OUT, ROWS_PER_TILE_OUT)],
                    out_hbm.at[core, pl.ds(sub * ROWS_PER_TILE_OUT,
                                           ROWS_PER_TILE_OUT), :])


def _sc_scatter(htab2d, src_p, et_p, tgt_p):
    mesh = plsc.VectorSubcoreMesh(core_axis_name="c", subcore_axis_name="s",
                                  num_cores=NC, num_subcores=NS)
    kfn = pl.kernel(
        _sc_body,
        out_type=jax.ShapeDtypeStruct((NC, N, D), jnp.float32),
        mesh=mesh,
        scratch_types=[
            pltpu.VMEM((C, K), jnp.int32),
            pltpu.VMEM((C, K), jnp.int32),
            pltpu.VMEM((C, K), jnp.int32),
            pltpu.VMEM((2, K, D), jnp.float32),
            pltpu.VMEM_SHARED((NPAD, D), jnp.float32),
            pltpu.SemaphoreType.DMA((2,)),
            pltpu.SemaphoreType.DMA((2,)),
        ],
    )
    return kfn(htab2d, src_p, et_p, tgt_p)


def _combine_body(h_ref, p_ref, out_ref):
    o = h_ref[:, 0, :] + p_ref[0] + p_ref[1]
    out_ref[...] = jnp.maximum(o, 0.0)


def _combine(htab, partials):
    return pl.pallas_call(
        _combine_body,
        grid=(GRID_N,),
        in_specs=[
            pl.BlockSpec((BN, 1, D), lambda i: (i, 0, 0)),
            pl.BlockSpec((NC, BN, D), lambda i: (0, i, 0)),
        ],
        out_specs=pl.BlockSpec((BN, D), lambda i: (i, 0)),
        out_shape=jax.ShapeDtypeStruct((N, D), jnp.float32),
    )(htab, partials)


def kernel(node_features, edge_index, edge_type, W_self, b_self, bases,
           coefficients):
    htab = _build_table(node_features, W_self, b_self, bases, coefficients)

    src = edge_index[0].astype(jnp.int32)
    tgt = edge_index[1].astype(jnp.int32)
    et = edge_type.astype(jnp.int32)
    pad = EP - E
    src_p = jnp.concatenate([src, jnp.zeros((pad,), jnp.int32)]).reshape(EP // K, K)
    et_p = jnp.concatenate([et, jnp.zeros((pad,), jnp.int32)]).reshape(EP // K, K)
    tgt_p = jnp.concatenate([tgt, jnp.full((pad,), N, jnp.int32)]).reshape(EP // K, K)

    partials = _sc_scatter(htab.reshape(N * NTAB, D), src_p, et_p, tgt_p)
    return _combine(htab, partials)


# speedup vs baseline: 4.4867x; 4.4867x over previous
import functools

import jax
import jax.numpy as jnp
from jax import lax
from jax.experimental import pallas as pl
from jax.experimental.pallas import tpu as pltpu
from jax.experimental.pallas import tpu_sc as plsc

N = 10000
E = 160000
D = 128
N_REL = 13
NTAB = N_REL

NC = 2
NS = 16
NW = NC * NS
K = 128
EPT = 5120
EP = NW * EPT
C = EPT // K
NPAD = 10240
ROWS_PER_TILE_ACC = NPAD // NS
ROWS_PER_TILE_OUT = N // NS

BN = 400
GRID_N = N // BN


def _table_body(x_ref, w_ref, b_ref, bases_ref, coef_ref, self_ref, tab_ref):
    x = x_ref[...]
    self_ref[...] = (
        jnp.dot(x, w_ref[...], preferred_element_type=jnp.float32) + b_ref[...]
    )
    h0 = jnp.dot(x, bases_ref[0], preferred_element_type=jnp.float32)
    h1 = jnp.dot(x, bases_ref[1], preferred_element_type=jnp.float32)
    for r in range(N_REL):
        tab_ref[:, r, :] = coef_ref[r, 0] * h0 + coef_ref[r, 1] * h1


def _build_table(x, W_self, b_self, bases, coefficients):
    return pl.pallas_call(
        _table_body,
        grid=(GRID_N,),
        in_specs=[
            pl.BlockSpec((BN, D), lambda i: (i, 0)),
            pl.BlockSpec((D, D), lambda i: (0, 0)),
            pl.BlockSpec((1, D), lambda i: (0, 0)),
            pl.BlockSpec((2, D, D), lambda i: (0, 0, 0)),
            pl.BlockSpec(memory_space=pltpu.SMEM),
        ],
        out_specs=[
            pl.BlockSpec((BN, D), lambda i: (i, 0)),
            pl.BlockSpec((BN, NTAB, D), lambda i: (i, 0, 0)),
        ],
        out_shape=[
            jax.ShapeDtypeStruct((N, D), jnp.float32),
            jax.ShapeDtypeStruct((N, NTAB, D), jnp.float32),
        ],
    )(x, W_self, b_self.reshape(1, D), bases, coefficients)


def _sc_body(htab_hbm, src_hbm, et_hbm, tgt_hbm, out_hbm,
             idx_v, tmp_v, tgt_v, rows_v, acc, sem_g, sem_s):
    core = lax.axis_index("c")
    sub = lax.axis_index("s")
    wid = core * NS + sub

    @pl.loop(0, K)
    def _(i):
        for j in range(D // 16):
            rows_v[0, i, pl.ds(j * 16, 16)] = jnp.zeros((16,), jnp.float32)

    for k in range(ROWS_PER_TILE_ACC // K):
        pltpu.sync_copy(rows_v.at[0],
                        acc.at[pl.ds(sub * ROWS_PER_TILE_ACC + k * K, K)])

    pltpu.sync_copy(src_hbm.at[pl.ds(wid * C, C)], idx_v)
    pltpu.sync_copy(et_hbm.at[pl.ds(wid * C, C)], tmp_v)
    pltpu.sync_copy(tgt_hbm.at[pl.ds(wid * C, C)], tgt_v)

    @pl.loop(0, C)
    def _(c):
        for j in range(K // 16):
            s = idx_v[c, pl.ds(j * 16, 16)]
            t = tmp_v[c, pl.ds(j * 16, 16)]
            idx_v[c, pl.ds(j * 16, 16)] = s * NTAB + t

    plsc.subcore_barrier()

    @pl.loop(0, C)
    def _(c):
        pltpu.async_copy(htab_hbm.at[idx_v.at[c]], rows_v.at[0],
                         sem_g.at[0]).wait()
        pltpu.async_copy(rows_v.at[0], acc.at[tgt_v.at[c]],
                         sem_s.at[0], add=True).wait()

    plsc.subcore_barrier()

    @pl.when(sub < NS - 1)
    def _():
        pltpu.sync_copy(acc.at[pl.ds(sub * 624, 624)],
                        out_hbm.at[core, pl.ds(sub * 624, 624), :])

    @pl.when(sub == NS - 1)
    def _():
        pltpu.sync_copy(acc.at[pl.ds((NS - 1) * 624, N - (NS - 1) * 624)],
                        out_hbm.at[core, pl.ds((NS - 1) * 624,
                                               N - (NS - 1) * 624), :])


def _sc_scatter(htab2d, src_p, et_p, tgt_p):
    mesh = plsc.VectorSubcoreMesh(core_axis_name="c", subcore_axis_name="s",
                                  num_cores=NC, num_subcores=NS)
    kfn = pl.kernel(
        _sc_body,
        out_type=jax.ShapeDtypeStruct((NC, N, D), jnp.float32),
        mesh=mesh,
        scratch_types=[
            pltpu.VMEM((C, K), jnp.int32),
            pltpu.VMEM((C, K), jnp.int32),
            pltpu.VMEM((C, K), jnp.int32),
            pltpu.VMEM((2, K, D), jnp.float32),
            pltpu.VMEM_SHARED((NPAD, D), jnp.float32),
            pltpu.SemaphoreType.DMA((2,)),
            pltpu.SemaphoreType.DMA((2,)),
        ],
    )
    return kfn(htab2d, src_p, et_p, tgt_p)


def _combine_body(h_ref, p_ref, out_ref):
    o = h_ref[...] + p_ref[0] + p_ref[1]
    out_ref[...] = jnp.maximum(o, 0.0)


def _combine(out_self, partials):
    return pl.pallas_call(
        _combine_body,
        grid=(GRID_N,),
        in_specs=[
            pl.BlockSpec((BN, D), lambda i: (i, 0)),
            pl.BlockSpec((NC, BN, D), lambda i: (0, i, 0)),
        ],
        out_specs=pl.BlockSpec((BN, D), lambda i: (i, 0)),
        out_shape=jax.ShapeDtypeStruct((N, D), jnp.float32),
    )(out_self, partials)


def kernel(node_features, edge_index, edge_type, W_self, b_self, bases,
           coefficients):
    out_self, htab = _build_table(node_features, W_self, b_self, bases,
                                  coefficients)

    src = edge_index[0].astype(jnp.int32)
    tgt = edge_index[1].astype(jnp.int32)
    et = edge_type.astype(jnp.int32)
    pad = EP - E
    src_p = jnp.concatenate([src, jnp.zeros((pad,), jnp.int32)]).reshape(EP // K, K)
    et_p = jnp.concatenate([et, jnp.zeros((pad,), jnp.int32)]).reshape(EP // K, K)
    tgt_p = jnp.concatenate([tgt, jnp.full((pad,), N, jnp.int32)]).reshape(EP // K, K)

    partials = _sc_scatter(htab.reshape(N * NTAB, D), src_p, et_p, tgt_p)
    return _combine(out_self, partials)


# speedup vs baseline: 5.3132x; 1.1842x over previous
import functools

import jax
import jax.numpy as jnp
from jax import lax
from jax.experimental import pallas as pl
from jax.experimental.pallas import tpu as pltpu
from jax.experimental.pallas import tpu_sc as plsc

N = 10000
E = 160000
D = 128
N_REL = 13
NTAB = N_REL

NC = 2
NS = 16
NW = NC * NS
K = 128
EPT = 5120
EP = NW * EPT
C = EPT // K
NBUF = 2
NPAD = 10112
ROWS_PER_TILE_ACC = NPAD // NS
ROWS_PER_TILE_OUT = N // NS

BN = 400
GRID_N = N // BN


def _table_body(x_ref, w_ref, b_ref, bases_ref, coef_ref, self_ref, tab_ref):
    x = x_ref[...]
    self_ref[...] = (
        jnp.dot(x, w_ref[...], preferred_element_type=jnp.float32) + b_ref[...]
    )
    h0 = jnp.dot(x, bases_ref[0], preferred_element_type=jnp.float32)
    h1 = jnp.dot(x, bases_ref[1], preferred_element_type=jnp.float32)
    for r in range(N_REL):
        tab_ref[r] = coef_ref[r, 0] * h0 + coef_ref[r, 1] * h1


def _build_table(x, W_self, b_self, bases, coefficients):
    return pl.pallas_call(
        _table_body,
        grid=(GRID_N,),
        in_specs=[
            pl.BlockSpec((BN, D), lambda i: (i, 0)),
            pl.BlockSpec((D, D), lambda i: (0, 0)),
            pl.BlockSpec((1, D), lambda i: (0, 0)),
            pl.BlockSpec((2, D, D), lambda i: (0, 0, 0)),
            pl.BlockSpec(memory_space=pltpu.SMEM),
        ],
        out_specs=[
            pl.BlockSpec((BN, D), lambda i: (i, 0)),
            pl.BlockSpec((NTAB, BN, D), lambda i: (0, i, 0)),
        ],
        out_shape=[
            jax.ShapeDtypeStruct((N, D), jnp.float32),
            jax.ShapeDtypeStruct((NTAB, N, D), jnp.float32),
        ],
    )(x, W_self, b_self.reshape(1, D), bases, coefficients)


def _sc_body(htab_hbm, src_hbm, et_hbm, tgt_hbm, out_hbm,
             idx_v, tmp_v, tgt_v, rows_v, acc, sem_g, sem_s):
    core = lax.axis_index("c")
    sub = lax.axis_index("s")
    wid = core * NS + sub

    @pl.loop(0, K)
    def _(i):
        for j in range(D // 16):
            rows_v[0, i, pl.ds(j * 16, 16)] = jnp.zeros((16,), jnp.float32)

    for k in range(4):
        pltpu.sync_copy(rows_v.at[0],
                        acc.at[pl.ds(sub * ROWS_PER_TILE_ACC + k * K, K)])
    pltpu.sync_copy(rows_v.at[0, pl.ds(0, ROWS_PER_TILE_ACC - 4 * K)],
                    acc.at[pl.ds(sub * ROWS_PER_TILE_ACC + 4 * K,
                                 ROWS_PER_TILE_ACC - 4 * K)])

    pltpu.sync_copy(src_hbm.at[pl.ds(wid * C, C)], idx_v)
    pltpu.sync_copy(et_hbm.at[pl.ds(wid * C, C)], tmp_v)
    pltpu.sync_copy(tgt_hbm.at[pl.ds(wid * C, C)], tgt_v)

    @pl.loop(0, C)
    def _(c):
        for j in range(K // 16):
            s = idx_v[c, pl.ds(j * 16, 16)]
            t = tmp_v[c, pl.ds(j * 16, 16)]
            idx_v[c, pl.ds(j * 16, 16)] = t * N + s

    plsc.subcore_barrier()

    pltpu.async_copy(htab_hbm.at[idx_v.at[0]], rows_v.at[0], sem_g.at[0])

    @pl.loop(0, C, step=NBUF)
    def _(c0):
        for b in range(NBUF):
            c = c0 + b
            b1 = (b + 1) % NBUF
            pltpu.make_async_copy(htab_hbm.at[idx_v.at[c]], rows_v.at[b],
                                  sem_g.at[b]).wait()
            pltpu.async_copy(rows_v.at[b], acc.at[tgt_v.at[c]],
                             sem_s.at[b], add=True)

            @pl.when(c >= 1)
            def _():
                pltpu.make_async_copy(rows_v.at[b1], acc.at[tgt_v.at[c - 1]],
                                      sem_s.at[b1]).wait()

            @pl.when(c + 1 < C)
            def _():
                pltpu.async_copy(htab_hbm.at[idx_v.at[c + 1]], rows_v.at[b1],
                                 sem_g.at[b1])

    pltpu.make_async_copy(rows_v.at[(C - 1) % NBUF], acc.at[tgt_v.at[C - 1]],
                          sem_s.at[(C - 1) % NBUF]).wait()

    plsc.subcore_barrier()

    @pl.when(sub < NS - 1)
    def _():
        pltpu.sync_copy(acc.at[pl.ds(sub * 624, 624)],
                        out_hbm.at[core, pl.ds(sub * 624, 624), :])

    @pl.when(sub == NS - 1)
    def _():
        pltpu.sync_copy(acc.at[pl.ds((NS - 1) * 624, N - (NS - 1) * 624)],
                        out_hbm.at[core, pl.ds((NS - 1) * 624,
                                               N - (NS - 1) * 624), :])


def _sc_scatter(htab2d, src_p, et_p, tgt_p):
    mesh = plsc.VectorSubcoreMesh(core_axis_name="c", subcore_axis_name="s",
                                  num_cores=NC, num_subcores=NS)
    kfn = pl.kernel(
        _sc_body,
        out_type=jax.ShapeDtypeStruct((NC, N, D), jnp.float32),
        mesh=mesh,
        scratch_types=[
            pltpu.VMEM((C, K), jnp.int32),
            pltpu.VMEM((C, K), jnp.int32),
            pltpu.VMEM((C, K), jnp.int32),
            pltpu.VMEM((NBUF, K, D), jnp.float32),
            pltpu.VMEM_SHARED((NPAD, D), jnp.float32),
            pltpu.SemaphoreType.DMA((NBUF,)),
            pltpu.SemaphoreType.DMA((NBUF,)),
        ],
    )
    return kfn(htab2d, src_p, et_p, tgt_p)


def _combine_body(h_ref, p_ref, out_ref):
    o = h_ref[...] + p_ref[0] + p_ref[1]
    out_ref[...] = jnp.maximum(o, 0.0)


def _combine(out_self, partials):
    return pl.pallas_call(
        _combine_body,
        grid=(GRID_N,),
        in_specs=[
            pl.BlockSpec((BN, D), lambda i: (i, 0)),
            pl.BlockSpec((NC, BN, D), lambda i: (0, i, 0)),
        ],
        out_specs=pl.BlockSpec((BN, D), lambda i: (i, 0)),
        out_shape=jax.ShapeDtypeStruct((N, D), jnp.float32),
    )(out_self, partials)


def kernel(node_features, edge_index, edge_type, W_self, b_self, bases,
           coefficients):
    out_self, htab = _build_table(node_features, W_self, b_self, bases,
                                  coefficients)

    src = edge_index[0].astype(jnp.int32)
    tgt = edge_index[1].astype(jnp.int32)
    et = edge_type.astype(jnp.int32)
    pad = EP - E
    src_p = jnp.concatenate([src, jnp.zeros((pad,), jnp.int32)]).reshape(EP // K, K)
    et_p = jnp.concatenate([et, jnp.zeros((pad,), jnp.int32)]).reshape(EP // K, K)
    tgt_p = jnp.concatenate([tgt, jnp.full((pad,), N, jnp.int32)]).reshape(EP // K, K)

    partials = _sc_scatter(htab.reshape(N * NTAB, D), src_p, et_p, tgt_p)
    return _combine(out_self, partials)


# speedup vs baseline: 5.8988x; 1.1102x over previous
import functools

import jax
import jax.numpy as jnp
from jax import lax
from jax.experimental import pallas as pl
from jax.experimental.pallas import tpu as pltpu
from jax.experimental.pallas import tpu_sc as plsc

N = 10000
E = 160000
D = 128
N_REL = 13
NTAB = N_REL

NC = 2
NS = 16
NW = NC * NS
K = 128
NCHUNK = 1280
EP = NCHUNK * K
CBIG = 64
CSMALL = 16
NBUF = 2
NPAD = 10112
ROWS_PER_TILE_ACC = NPAD // NS
ROWS_PER_TILE_OUT = N // NS

BN = 400
GRID_N = N // BN


def _table_body(x_ref, w_ref, b_ref, bases_ref, coef_ref, self_ref, tab_ref):
    x = x_ref[...]
    self_ref[...] = (
        jnp.dot(x, w_ref[...], preferred_element_type=jnp.float32) + b_ref[...]
    )
    h0 = jnp.dot(x, bases_ref[0], preferred_element_type=jnp.float32)
    h1 = jnp.dot(x, bases_ref[1], preferred_element_type=jnp.float32)
    for r in range(N_REL):
        tab_ref[r] = coef_ref[r, 0] * h0 + coef_ref[r, 1] * h1


def _build_table(x, W_self, b_self, bases, coefficients):
    return pl.pallas_call(
        _table_body,
        grid=(GRID_N,),
        in_specs=[
            pl.BlockSpec((BN, D), lambda i: (i, 0)),
            pl.BlockSpec((D, D), lambda i: (0, 0)),
            pl.BlockSpec((1, D), lambda i: (0, 0)),
            pl.BlockSpec((2, D, D), lambda i: (0, 0, 0)),
            pl.BlockSpec(memory_space=pltpu.SMEM),
        ],
        out_specs=[
            pl.BlockSpec((BN, D), lambda i: (i, 0)),
            pl.BlockSpec((NTAB, BN, D), lambda i: (0, i, 0)),
        ],
        out_shape=[
            jax.ShapeDtypeStruct((N, D), jnp.float32),
            jax.ShapeDtypeStruct((NTAB, N, D), jnp.float32),
        ],
    )(x, W_self, b_self.reshape(1, D), bases, coefficients)


def _sc_body(htab_hbm, src_hbm, et_hbm, tgt_hbm, out_hbm,
             idx_v, tgt_v, rows_v, acc, sem_g, sem_s):
    core = lax.axis_index("c")
    sub = lax.axis_index("s")

    @pl.loop(0, K)
    def _(i):
        for j in range(D // 16):
            rows_v[0, i, pl.ds(j * 16, 16)] = jnp.zeros((16,), jnp.float32)

    for k in range(ROWS_PER_TILE_ACC // K):
        pltpu.sync_copy(rows_v.at[0],
                        acc.at[pl.ds(sub * ROWS_PER_TILE_ACC + k * K, K)])
    rem = ROWS_PER_TILE_ACC % K
    pltpu.sync_copy(rows_v.at[0, pl.ds(0, rem)],
                    acc.at[pl.ds(sub * ROWS_PER_TILE_ACC
                                 + (ROWS_PER_TILE_ACC // K) * K, rem)])

    m = jnp.where(core == 0, CBIG, CSMALL)
    base = jnp.where(core == 0, sub * CBIG, NS * CBIG + sub * CSMALL)

    @pl.when(core == 0)
    def _():
        pltpu.sync_copy(src_hbm.at[pl.ds(sub * CBIG, CBIG)],
                        idx_v.at[pl.ds(0, CBIG)])
        pltpu.sync_copy(et_hbm.at[pl.ds(sub * CBIG, CBIG)],
                        tgt_v.at[pl.ds(0, CBIG)])

    @pl.when(core == 1)
    def _():
        pltpu.sync_copy(src_hbm.at[pl.ds(NS * CBIG + sub * CSMALL, CSMALL)],
                        idx_v.at[pl.ds(0, CSMALL)])
        pltpu.sync_copy(et_hbm.at[pl.ds(NS * CBIG + sub * CSMALL, CSMALL)],
                        tgt_v.at[pl.ds(0, CSMALL)])

    @pl.loop(0, m)
    def _(c):
        for j in range(K // 16):
            s = idx_v[c, pl.ds(j * 16, 16)]
            t = tgt_v[c, pl.ds(j * 16, 16)]
            idx_v[c, pl.ds(j * 16, 16)] = t * N + s

    @pl.when(core == 0)
    def _():
        pltpu.sync_copy(tgt_hbm.at[pl.ds(sub * CBIG, CBIG)],
                        tgt_v.at[pl.ds(0, CBIG)])

    @pl.when(core == 1)
    def _():
        pltpu.sync_copy(tgt_hbm.at[pl.ds(NS * CBIG + sub * CSMALL, CSMALL)],
                        tgt_v.at[pl.ds(0, CSMALL)])

    plsc.subcore_barrier()

    pltpu.async_copy(htab_hbm.at[idx_v.at[0]], rows_v.at[0], sem_g.at[0])

    @pl.loop(0, m, step=NBUF)
    def _(c0):
        for b in range(NBUF):
            c = c0 + b
            b1 = (b + 1) % NBUF
            pltpu.make_async_copy(htab_hbm.at[idx_v.at[c]], rows_v.at[b],
                                  sem_g.at[b]).wait()
            pltpu.async_copy(rows_v.at[b], acc.at[tgt_v.at[c]],
                             sem_s.at[b], add=True)

            @pl.when(c >= 1)
            def _():
                pltpu.make_async_copy(rows_v.at[b1], acc.at[tgt_v.at[c - 1]],
                                      sem_s.at[b1]).wait()

            @pl.when(c + 1 < m)
            def _():
                pltpu.async_copy(htab_hbm.at[idx_v.at[c + 1]], rows_v.at[b1],
                                 sem_g.at[b1])

    pltpu.make_async_copy(rows_v.at[1], acc.at[tgt_v.at[m - 1]],
                          sem_s.at[1]).wait()

    plsc.subcore_barrier()

    @pl.when(sub < NS - 1)
    def _():
        pltpu.sync_copy(acc.at[pl.ds(sub * 624, 624)],
                        out_hbm.at[core, pl.ds(sub * 624, 624), :])

    @pl.when(sub == NS - 1)
    def _():
        pltpu.sync_copy(acc.at[pl.ds((NS - 1) * 624, N - (NS - 1) * 624)],
                        out_hbm.at[core, pl.ds((NS - 1) * 624,
                                               N - (NS - 1) * 624), :])


def _sc_scatter(htab2d, src_p, et_p, tgt_p):
    mesh = plsc.VectorSubcoreMesh(core_axis_name="c", subcore_axis_name="s",
                                  num_cores=NC, num_subcores=NS)
    kfn = pl.kernel(
        _sc_body,
        out_type=jax.ShapeDtypeStruct((NC, N, D), jnp.float32),
        mesh=mesh,
        scratch_types=[
            pltpu.VMEM((CBIG, K), jnp.int32),
            pltpu.VMEM((CBIG, K), jnp.int32),
            pltpu.VMEM((NBUF, K, D), jnp.float32),
            pltpu.VMEM_SHARED((NPAD, D), jnp.float32),
            pltpu.SemaphoreType.DMA((NBUF,)),
            pltpu.SemaphoreType.DMA((NBUF,)),
        ],
    )
    return kfn(htab2d, src_p, et_p, tgt_p)


def _combine_body(h_ref, p_ref, out_ref):
    o = h_ref[...] + p_ref[0] + p_ref[1]
    out_ref[...] = jnp.maximum(o, 0.0)


def _combine(out_self, partials):
    return pl.pallas_call(
        _combine_body,
        grid=(GRID_N,),
        in_specs=[
            pl.BlockSpec((BN, D), lambda i: (i, 0)),
            pl.BlockSpec((NC, BN, D), lambda i: (0, i, 0)),
        ],
        out_specs=pl.BlockSpec((BN, D), lambda i: (i, 0)),
        out_shape=jax.ShapeDtypeStruct((N, D), jnp.float32),
    )(out_self, partials)


def kernel(node_features, edge_index, edge_type, W_self, b_self, bases,
           coefficients):
    out_self, htab = _build_table(node_features, W_self, b_self, bases,
                                  coefficients)

    src = edge_index[0].astype(jnp.int32)
    tgt = edge_index[1].astype(jnp.int32)
    et = edge_type.astype(jnp.int32)
    pad = EP - E
    src_p = jnp.concatenate([src, jnp.zeros((pad,), jnp.int32)]).reshape(EP // K, K)
    et_p = jnp.concatenate([et, jnp.zeros((pad,), jnp.int32)]).reshape(EP // K, K)
    tgt_p = jnp.concatenate([tgt, jnp.full((pad,), N, jnp.int32)]).reshape(EP // K, K)

    partials = _sc_scatter(htab.reshape(N * NTAB, D), src_p, et_p, tgt_p)
    return _combine(out_self, partials)
